# baseline (speedup 1.0000x reference)
import functools
from math import factorial

import jax
import jax.numpy as jnp
import numpy as np
from jax.experimental import pallas as pl

_NZ = 4
_NAB = 2
_CH = _NAB * _NAB
_CUTOFF = 5.5
_NRBF = 6
_NRB = 8
_MAXL = 2
_AVG = 16.0
_LXLYLZ = [(0, 0, 0), (1, 0, 0), (0, 1, 0), (0, 0, 1), (2, 0, 0), (1, 1, 0),
           (1, 0, 1), (0, 2, 0), (0, 1, 1), (0, 0, 2)]
_NANG = len(_LXLYLZ)
_LOFA = np.array([lx + ly + lz for (lx, ly, lz) in _LXLYLZ])


def _edge_basis_body(vec_ref, arw_ref, rr_ref, ang_ref, filt_ref, rcut_ref):
    vec = vec_ref[...]
    x, y, z = vec[0], vec[1], vec[2]
    l2 = x * x + y * y + z * z
    lengths = jnp.sqrt(l2)
    inv = 1.0 / (lengths + 1e-9)
    ux, uy, uz = x * inv, y * inv, z * inv

    r = lengths / _CUTOFF
    p = 6.0
    r6 = r ** 6
    env = (1.0 - 0.5 * (p + 1) * (p + 2) * r6 + p * (p + 2) * r6 * r
           - 0.5 * p * (p + 1) * r6 * r * r)
    rcut = env * (lengths < _CUTOFF).astype(jnp.float32)
    rcut_ref[...] = rcut[None, :]

    scale = jnp.sqrt(2.0 / _CUTOFF)
    invl = 1.0 / (lengths + 1e-9)
    bess = []
    for n in range(1, _NRB + 1):
        bess.append(scale * jnp.sin(n * jnp.pi * lengths / _CUTOFF) * invl)
    bess = jnp.stack(bess, axis=0)
    rr_ref[...] = bess[:_NRBF] * rcut[None, :]

    angs = []
    for (lx, ly, lz) in _LXLYLZ:
        v = jnp.ones_like(ux)
        for _ in range(lx):
            v = v * ux
        for _ in range(ly):
            v = v * uy
        for _ in range(lz):
            v = v * uz
        angs.append(v)
    ang_ref[...] = jnp.stack(angs, axis=0)

    arw = arw_ref[...]
    filt = jax.lax.dot_general(arw, bess, (((0,), (0,)), ((), ())),
                               preferred_element_type=jnp.float32)
    filt_ref[...] = filt * rcut[None, :]


def _edge_basis(vec_t, ar_w, n_edges):
    BE = 3200
    grid = (n_edges // BE,)
    out_shapes = (
        jax.ShapeDtypeStruct((_NRBF, n_edges), jnp.float32),
        jax.ShapeDtypeStruct((_NANG, n_edges), jnp.float32),
        jax.ShapeDtypeStruct((_NRB, n_edges), jnp.float32),
        jax.ShapeDtypeStruct((1, n_edges), jnp.float32),
    )
    return pl.pallas_call(
        _edge_basis_body,
        grid=grid,
        in_specs=[
            pl.BlockSpec((3, BE), lambda i: (0, i)),
            pl.BlockSpec((_NRB, _NRB), lambda i: (0, 0)),
        ],
        out_specs=(
            pl.BlockSpec((_NRBF, BE), lambda i: (0, i)),
            pl.BlockSpec((_NANG, BE), lambda i: (0, i)),
            pl.BlockSpec((_NRB, BE), lambda i: (0, i)),
            pl.BlockSpec((1, BE), lambda i: (0, i)),
        ),
        out_shape=out_shapes,
    )(vec_t, ar_w)


def _radial_transform(A, rt_w):
    return jnp.einsum('nrac,ard->ndac', A, rt_w[_LOFA])


def _symmetrizer(A):
    feats = [A[:, :, 0, :]]
    for l in range(_MAXL + 1):
        acc = jnp.zeros_like(A[:, :, 0, :])
        for a, (lx, ly, lz) in enumerate(_LXLYLZ):
            if lx + ly + lz == l:
                c = float(factorial(l) / (factorial(lx) * factorial(ly) * factorial(lz)))
                acc = acc + c * A[:, :, a, :] ** 2
        feats.append(acc)
    return jnp.stack(feats, axis=2)


def kernel(positions, atomic_numbers, edge_index, shifts, batch, cell,
           emb_w, rt_w, nm_w, ar_w, bchi_w):
    n_nodes = positions.shape[0]
    n_edges = edge_index.shape[1]
    onehot = jax.nn.one_hot(atomic_numbers, _NZ, dtype=positions.dtype)
    node_emb = onehot @ emb_w
    snd = edge_index[0]
    rcv = edge_index[1]
    enc = (node_emb[snd][:, :, None] * node_emb[rcv][:, None, :]).reshape(-1, _CH)
    vec = positions[rcv] - positions[snd] + shifts
    vec_t = vec.T

    rr_t, ang_t, filt_t, rcut_t = _edge_basis(vec_t, ar_w, n_edges)
    rr = rr_t.T
    ang = ang_t.T
    filt = filt_t.T

    edge_attri = jnp.einsum('er,ea,ec->erac', rr, ang, enc)
    A0 = jnp.zeros((n_nodes, _NRBF, _NANG, _CH), positions.dtype).at[rcv].add(edge_attri)
    A = _radial_transform(A0, rt_w)
    B = _symmetrizer(A)
    feats = [B]
    mpn = 1.0 / _AVG ** 0.5

    memory = A * jnp.transpose(nm_w[_LOFA], (1, 0, 2))[None]
    scal = jnp.einsum('rb,erbc->ec', bchi_w, B[snd])
    msg_bchi = scal[:, None, None, :] * edge_attri
    A_bchi = jnp.zeros((n_nodes, _NRBF, _NANG, _CH), positions.dtype).at[rcv].add(msg_bchi)
    A_bchi = _radial_transform(A_bchi, rt_w)
    msg_ar = A[snd] * filt[:, :, None, None]
    A_ar = jnp.zeros_like(A).at[rcv].add(msg_ar)
    A2 = (A_ar + A_bchi) * mpn + memory
    B2 = _symmetrizer(A2)
    feats.append(B2)
    node_feats = jnp.stack(feats, axis=-1)
    return node_feats


# speedup vs baseline: 1.0517x; 1.0517x over previous
import functools
from math import factorial

import jax
import jax.numpy as jnp
import numpy as np
from jax import lax
from jax.experimental import pallas as pl
from jax.experimental.pallas import tpu as pltpu
from jax.experimental.pallas import tpu_sc as plsc

_NZ = 4
_NAB = 2
_CH = _NAB * _NAB
_CUTOFF = 5.5
_NRBF = 6
_NRB = 8
_MAXL = 2
_AVG = 16.0
_LXLYLZ = [(0, 0, 0), (1, 0, 0), (0, 1, 0), (0, 0, 1), (2, 0, 0), (1, 1, 0),
           (1, 0, 1), (0, 2, 0), (0, 1, 1), (0, 0, 2)]
_NANG = len(_LXLYLZ)
_LOFA = np.array([lx + ly + lz for (lx, ly, lz) in _LXLYLZ])

_N = 10000
_E = 160000
_E_PAD = 163840
_NC = 2
_NS = 16
_NW = _NC * _NS
_EPW = _E_PAD // _NW

_mesh = plsc.VectorSubcoreMesh(core_axis_name="c", subcore_axis_name="s")


def _f16(v):
    return jnp.full((16,), v, jnp.int32)


def _p0_body(pos_hbm, emb_hbm, ei_hbm, ev_hbm, pos_v, emb_v, snd_v, rcv_v, out_v):
    cid = lax.axis_index("c")
    sid = lax.axis_index("s")
    w = sid * _NC + cid
    base = w * _EPW
    pltpu.sync_copy(pos_hbm, pos_v)
    pltpu.sync_copy(emb_hbm, emb_v)
    pltpu.sync_copy(ei_hbm.at[pl.ds(base, _EPW)], snd_v)
    pltpu.sync_copy(ei_hbm.at[pl.ds(_E_PAD + base, _EPW)], rcv_v)

    n1 = _f16(_N)
    n2 = _f16(2 * _N)

    def body(g, carry):
        j0 = g * 16
        snd16 = snd_v[pl.ds(j0, 16)]
        rcv16 = rcv_v[pl.ds(j0, 16)]
        pxs = plsc.load_gather(pos_v, [snd16])
        pys = plsc.load_gather(pos_v, [snd16 + n1])
        pzs = plsc.load_gather(pos_v, [snd16 + n2])
        pxr = plsc.load_gather(pos_v, [rcv16])
        pyr = plsc.load_gather(pos_v, [rcv16 + n1])
        pzr = plsc.load_gather(pos_v, [rcv16 + n2])
        es0 = plsc.load_gather(emb_v, [snd16])
        es1 = plsc.load_gather(emb_v, [snd16 + n1])
        er0 = plsc.load_gather(emb_v, [rcv16])
        er1 = plsc.load_gather(emb_v, [rcv16 + n1])
        out_v[pl.ds(0 * _EPW + j0, 16)] = pxr - pxs
        out_v[pl.ds(1 * _EPW + j0, 16)] = pyr - pys
        out_v[pl.ds(2 * _EPW + j0, 16)] = pzr - pzs
        out_v[pl.ds(3 * _EPW + j0, 16)] = es0 * er0
        out_v[pl.ds(4 * _EPW + j0, 16)] = es0 * er1
        out_v[pl.ds(5 * _EPW + j0, 16)] = es1 * er0
        out_v[pl.ds(6 * _EPW + j0, 16)] = es1 * er1
        return carry

    lax.fori_loop(0, _EPW // 16, body, 0)
    for comp in range(7):
        pltpu.sync_copy(out_v.at[pl.ds(comp * _EPW, _EPW)],
                        ev_hbm.at[pl.ds(comp * _E_PAD + base, _EPW)])


_sc_params = pltpu.CompilerParams(needs_layout_passes=False)

_p0 = functools.partial(
    pl.kernel,
    out_type=jax.ShapeDtypeStruct((7 * _E_PAD,), jnp.float32),
    mesh=_mesh,
    compiler_params=_sc_params,
    scratch_types=[
        pltpu.VMEM((3 * _N,), jnp.float32),
        pltpu.VMEM((2 * _N,), jnp.float32),
        pltpu.VMEM((_EPW,), jnp.int32),
        pltpu.VMEM((_EPW,), jnp.int32),
        pltpu.VMEM((7 * _EPW,), jnp.float32),
    ],
)(_p0_body)


def _p1_body(ev_ref, arw_ref, ef_ref, ft_ref):
    ev = ev_ref[...]
    x, y, z = ev[0], ev[1], ev[2]
    enc = ev[3:7]
    l2 = x * x + y * y + z * z
    lengths = jnp.sqrt(l2)
    inv = 1.0 / (lengths + 1e-9)
    ux, uy, uz = x * inv, y * inv, z * inv

    r = lengths / _CUTOFF
    p = 6.0
    r6 = r ** 6
    env = (1.0 - 0.5 * (p + 1) * (p + 2) * r6 + p * (p + 2) * r6 * r
           - 0.5 * p * (p + 1) * r6 * r * r)
    rcut = env * (lengths < _CUTOFF).astype(jnp.float32)

    scale = jnp.sqrt(2.0 / _CUTOFF)
    bess = []
    for n in range(1, _NRB + 1):
        bess.append(scale * jnp.sin(n * jnp.pi * lengths / _CUTOFF) * inv)
    bess = jnp.stack(bess, axis=0)
    rr = bess[:_NRBF] * rcut[None, :]

    angs = []
    for (lx, ly, lz) in _LXLYLZ:
        v = jnp.ones_like(ux)
        for _ in range(lx):
            v = v * ux
        for _ in range(ly):
            v = v * uy
        for _ in range(lz):
            v = v * uz
        angs.append(v)
    ang = jnp.stack(angs, axis=0)
    wfe = (ang[:, None, :] * enc[None, :, :]).reshape(40, ang.shape[-1])

    ef_ref[...] = jnp.concatenate(
        [rr, jnp.zeros((2, rr.shape[-1]), jnp.float32), wfe], axis=0)

    arw = arw_ref[...]
    filt = lax.dot_general(arw, bess, (((0,), (0,)), ((), ())),
                           preferred_element_type=jnp.float32)
    ft_ref[...] = filt * rcut[None, :]


def _p1(ev, ar_w):
    BE = 512
    grid = (_E_PAD // BE,)
    return pl.pallas_call(
        _p1_body,
        grid=grid,
        in_specs=[
            pl.BlockSpec((7, BE), lambda i: (0, i)),
            pl.BlockSpec((_NRB, _NRB), lambda i: (0, 0)),
        ],
        out_specs=(
            pl.BlockSpec((48, BE), lambda i: (0, i)),
            pl.BlockSpec((_NRB, BE), lambda i: (0, i)),
        ),
        out_shape=(
            jax.ShapeDtypeStruct((48, _E_PAD), jnp.float32),
            jax.ShapeDtypeStruct((_NRB, _E_PAD), jnp.float32),
        ),
    )(ev, ar_w)


def _radial_transform(A, rt_w):
    return jnp.einsum('nrac,ard->ndac', A, rt_w[_LOFA])


def _symmetrizer(A):
    feats = [A[:, :, 0, :]]
    for l in range(_MAXL + 1):
        acc = jnp.zeros_like(A[:, :, 0, :])
        for a, (lx, ly, lz) in enumerate(_LXLYLZ):
            if lx + ly + lz == l:
                c = float(factorial(l) / (factorial(lx) * factorial(ly) * factorial(lz)))
                acc = acc + c * A[:, :, a, :] ** 2
        feats.append(acc)
    return jnp.stack(feats, axis=2)


def kernel(positions, atomic_numbers, edge_index, shifts, batch, cell,
           emb_w, rt_w, nm_w, ar_w, bchi_w):
    n_nodes = positions.shape[0]
    onehot = jax.nn.one_hot(atomic_numbers, _NZ, dtype=positions.dtype)
    node_emb = onehot @ emb_w

    ei_p = jnp.zeros((2, _E_PAD), jnp.int32).at[:, :_E].set(edge_index).reshape(-1)
    pos_t = positions.T.reshape(-1)
    emb_t = node_emb.T.reshape(-1)

    ev = _p0(pos_t, emb_t, ei_p).reshape(7, _E_PAD)
    EF, FT = _p1(ev, ar_w)

    snd = edge_index[0]
    rcv = edge_index[1]
    rr_t = EF[0:6, :_E]
    w_t = EF[8:48, :_E]
    filt = FT[:, :_E].T

    edge_attri = jnp.einsum('re,we->erw', rr_t, w_t).reshape(_E, _NRBF, _NANG, _CH)
    A0 = jnp.zeros((n_nodes, _NRBF, _NANG, _CH), positions.dtype).at[rcv].add(edge_attri)
    A = _radial_transform(A0, rt_w)
    B = _symmetrizer(A)
    mpn = 1.0 / _AVG ** 0.5

    memory = A * jnp.transpose(nm_w[_LOFA], (1, 0, 2))[None]
    Bs = jnp.einsum('rb,nrbc->nc', bchi_w, B)
    scal = Bs[snd]
    msg_bchi = scal[:, None, None, :] * edge_attri
    A_bchi = jnp.zeros((n_nodes, _NRBF, _NANG, _CH), positions.dtype).at[rcv].add(msg_bchi)
    A_bchi = _radial_transform(A_bchi, rt_w)
    msg_ar = A[snd] * filt[:, :, None, None]
    A_ar = jnp.zeros_like(A).at[rcv].add(msg_ar)
    A2 = (A_ar + A_bchi) * mpn + memory
    B2 = _symmetrizer(A2)
    node_feats = jnp.stack([B, B2], axis=-1)
    return node_feats


# speedup vs baseline: 1.3509x; 1.2844x over previous
import functools
from math import factorial

import jax
import jax.numpy as jnp
import numpy as np
from jax import lax
from jax.experimental import pallas as pl
from jax.experimental.pallas import tpu as pltpu
from jax.experimental.pallas import tpu_sc as plsc

_NZ = 4
_NAB = 2
_CH = _NAB * _NAB
_CUTOFF = 5.5
_NRBF = 6
_NRB = 8
_MAXL = 2
_AVG = 16.0
_LXLYLZ = [(0, 0, 0), (1, 0, 0), (0, 1, 0), (0, 0, 1), (2, 0, 0), (1, 1, 0),
           (1, 0, 1), (0, 2, 0), (0, 1, 1), (0, 0, 2)]
_NANG = len(_LXLYLZ)
_LOFA = np.array([lx + ly + lz for (lx, ly, lz) in _LXLYLZ])

_N = 10000
_E = 160000
_E_PAD = 163840
_NC = 2
_NS = 16
_NW = _NC * _NS
_EPW = _E_PAD // _NW

_mesh = plsc.VectorSubcoreMesh(core_axis_name="c", subcore_axis_name="s")


def _f16(v):
    return jnp.full((16,), v, jnp.int32)


def _p0_body(pos_hbm, emb_hbm, ei_hbm, ev_hbm, pos_v, emb_v, snd_v, rcv_v, out_v):
    cid = lax.axis_index("c")
    sid = lax.axis_index("s")
    w = sid * _NC + cid
    base = w * _EPW
    pltpu.sync_copy(pos_hbm, pos_v)
    pltpu.sync_copy(emb_hbm, emb_v)
    pltpu.sync_copy(ei_hbm.at[pl.ds(base, _EPW)], snd_v)
    pltpu.sync_copy(ei_hbm.at[pl.ds(_E_PAD + base, _EPW)], rcv_v)

    n1 = _f16(_N)
    n2 = _f16(2 * _N)

    def body(g, carry):
        j0 = g * 16
        snd16 = snd_v[pl.ds(j0, 16)]
        rcv16 = rcv_v[pl.ds(j0, 16)]
        pxs = plsc.load_gather(pos_v, [snd16])
        pys = plsc.load_gather(pos_v, [snd16 + n1])
        pzs = plsc.load_gather(pos_v, [snd16 + n2])
        pxr = plsc.load_gather(pos_v, [rcv16])
        pyr = plsc.load_gather(pos_v, [rcv16 + n1])
        pzr = plsc.load_gather(pos_v, [rcv16 + n2])
        es0 = plsc.load_gather(emb_v, [snd16])
        es1 = plsc.load_gather(emb_v, [snd16 + n1])
        er0 = plsc.load_gather(emb_v, [rcv16])
        er1 = plsc.load_gather(emb_v, [rcv16 + n1])
        out_v[pl.ds(0 * _EPW + j0, 16)] = pxr - pxs
        out_v[pl.ds(1 * _EPW + j0, 16)] = pyr - pys
        out_v[pl.ds(2 * _EPW + j0, 16)] = pzr - pzs
        out_v[pl.ds(3 * _EPW + j0, 16)] = es0 * er0
        out_v[pl.ds(4 * _EPW + j0, 16)] = es0 * er1
        out_v[pl.ds(5 * _EPW + j0, 16)] = es1 * er0
        out_v[pl.ds(6 * _EPW + j0, 16)] = es1 * er1
        return carry

    lax.fori_loop(0, _EPW // 16, body, 0)
    for comp in range(7):
        pltpu.sync_copy(out_v.at[pl.ds(comp * _EPW, _EPW)],
                        ev_hbm.at[pl.ds(comp * _E_PAD + base, _EPW)])


_sc_params = pltpu.CompilerParams(needs_layout_passes=False)

_p0 = functools.partial(
    pl.kernel,
    out_type=jax.ShapeDtypeStruct((7 * _E_PAD,), jnp.float32),
    mesh=_mesh,
    compiler_params=_sc_params,
    scratch_types=[
        pltpu.VMEM((3 * _N,), jnp.float32),
        pltpu.VMEM((2 * _N,), jnp.float32),
        pltpu.VMEM((_EPW,), jnp.int32),
        pltpu.VMEM((_EPW,), jnp.int32),
        pltpu.VMEM((7 * _EPW,), jnp.float32),
    ],
)(_p0_body)


_K = 256
_EPT = _E_PAD // _NS
_NCHUNK = _EPT // _K
_N_PAD = 10240
_NPT = _N_PAD // _NS

_fl = np.arange(128)
_rpat_np = np.where(_fl < 120, _fl // 20, 6).astype(np.int32)
_wpat_np = np.where(_fl < 120, 8 + ((_fl % 20) // 2) * 4 + (_fl % 2), 6).astype(np.int32)
_wvalid_np = (_fl < 120).astype(np.int32)


def _pass1_body(ef_hbm, rcv_hbm, out_hbm, ef_v, rcv_v, rows_v, table):
    cid = lax.axis_index("c")
    tid = lax.axis_index("s")
    zf = jnp.zeros((16,), jnp.float32)

    def zrow(j, c):
        for i in range(8):
            rows_v[j, pl.ds(i * 16, 16)] = zf
        return c
    lax.fori_loop(0, 128, zrow, 0)
    for i in range(5):
        pltpu.sync_copy(rows_v.at[pl.ds(0, 128)],
                        table.at[pl.ds(tid * _NPT + i * 128, 128)])
    plsc.subcore_barrier()

    iota = lax.iota(jnp.int32, 16)
    rpats, wpats = [], []
    for i in range(8):
        fv = iota + 16 * i
        rpats.append(jnp.where(fv < 120, fv // 20, 6))
        wpats.append(jnp.where(fv < 120,
                               8 + ((fv % 20) // 2) * 4 + (fv % 2) + 2 * cid, 6))

    def chunk(ci, c):
        base = tid * _EPT + ci * _K
        pltpu.sync_copy(ef_hbm.at[:, pl.ds(base, _K)], ef_v)
        pltpu.sync_copy(rcv_hbm.at[pl.ds(base, _K)], rcv_v)

        def edge(j, c2):
            spl = jnp.full((16,), j, jnp.int32)
            for i in range(8):
                rrv = plsc.load_gather(ef_v, [rpats[i], spl])
                wv = plsc.load_gather(ef_v, [wpats[i], spl])
                rows_v[j, pl.ds(i * 16, 16)] = rrv * wv
            return c2
        lax.fori_loop(0, _K, edge, 0)
        pltpu.sync_copy(rows_v, table.at[rcv_v], add=True)
        return c
    lax.fori_loop(0, _NCHUNK, chunk, 0)
    plsc.subcore_barrier()
    pltpu.sync_copy(table.at[pl.ds(tid * _NPT, _NPT)],
                    out_hbm.at[cid, pl.ds(tid * _NPT, _NPT)])


_pass1 = functools.partial(
    pl.kernel,
    out_type=jax.ShapeDtypeStruct((_NC, _N_PAD, 128), jnp.float32),
    mesh=_mesh,
    compiler_params=_sc_params,
    scratch_types=[
        pltpu.VMEM((48, _K), jnp.float32),
        pltpu.VMEM((_K,), jnp.int32),
        pltpu.VMEM((_K, 128), jnp.float32),
        pltpu.VMEM_SHARED((_N_PAD, 128), jnp.float32),
    ],
)(_pass1_body)


def _p1_body(ev_ref, arw_ref, ef_ref, ft_ref):
    ev = ev_ref[...]
    x, y, z = ev[0], ev[1], ev[2]
    enc = ev[3:7]
    l2 = x * x + y * y + z * z
    lengths = jnp.sqrt(l2)
    inv = 1.0 / (lengths + 1e-9)
    ux, uy, uz = x * inv, y * inv, z * inv

    r = lengths / _CUTOFF
    p = 6.0
    r6 = r ** 6
    env = (1.0 - 0.5 * (p + 1) * (p + 2) * r6 + p * (p + 2) * r6 * r
           - 0.5 * p * (p + 1) * r6 * r * r)
    rcut = env * (lengths < _CUTOFF).astype(jnp.float32)

    scale = jnp.sqrt(2.0 / _CUTOFF)
    bess = []
    for n in range(1, _NRB + 1):
        bess.append(scale * jnp.sin(n * jnp.pi * lengths / _CUTOFF) * inv)
    bess = jnp.stack(bess, axis=0)
    rr = bess[:_NRBF] * rcut[None, :]

    angs = []
    for (lx, ly, lz) in _LXLYLZ:
        v = jnp.ones_like(ux)
        for _ in range(lx):
            v = v * ux
        for _ in range(ly):
            v = v * uy
        for _ in range(lz):
            v = v * uz
        angs.append(v)
    ang = jnp.stack(angs, axis=0)
    wfe = (ang[:, None, :] * enc[None, :, :]).reshape(40, ang.shape[-1])

    ef_ref[...] = jnp.concatenate(
        [rr, jnp.zeros((2, rr.shape[-1]), jnp.float32), wfe], axis=0)

    arw = arw_ref[...]
    filt = lax.dot_general(arw, bess, (((0,), (0,)), ((), ())),
                           preferred_element_type=jnp.float32)
    ft_ref[...] = filt * rcut[None, :]


def _p1(ev, ar_w):
    BE = 512
    grid = (_E_PAD // BE,)
    return pl.pallas_call(
        _p1_body,
        grid=grid,
        in_specs=[
            pl.BlockSpec((7, BE), lambda i: (0, i)),
            pl.BlockSpec((_NRB, _NRB), lambda i: (0, 0)),
        ],
        out_specs=(
            pl.BlockSpec((48, BE), lambda i: (0, i)),
            pl.BlockSpec((_NRB, BE), lambda i: (0, i)),
        ),
        out_shape=(
            jax.ShapeDtypeStruct((48, _E_PAD), jnp.float32),
            jax.ShapeDtypeStruct((_NRB, _E_PAD), jnp.float32),
        ),
    )(ev, ar_w)


def _radial_transform(A, rt_w):
    return jnp.einsum('nrac,ard->ndac', A, rt_w[_LOFA])


def _symmetrizer(A):
    feats = [A[:, :, 0, :]]
    for l in range(_MAXL + 1):
        acc = jnp.zeros_like(A[:, :, 0, :])
        for a, (lx, ly, lz) in enumerate(_LXLYLZ):
            if lx + ly + lz == l:
                c = float(factorial(l) / (factorial(lx) * factorial(ly) * factorial(lz)))
                acc = acc + c * A[:, :, a, :] ** 2
        feats.append(acc)
    return jnp.stack(feats, axis=2)


def kernel(positions, atomic_numbers, edge_index, shifts, batch, cell,
           emb_w, rt_w, nm_w, ar_w, bchi_w):
    n_nodes = positions.shape[0]
    onehot = jax.nn.one_hot(atomic_numbers, _NZ, dtype=positions.dtype)
    node_emb = onehot @ emb_w

    ei_p = jnp.zeros((2, _E_PAD), jnp.int32).at[:, :_E].set(edge_index).reshape(-1)
    pos_t = positions.T.reshape(-1)
    emb_t = node_emb.T.reshape(-1)

    ev = _p0(pos_t, emb_t, ei_p).reshape(7, _E_PAD)
    EF, FT = _p1(ev, ar_w)

    snd = edge_index[0]
    rcv = edge_index[1]
    rr_t = EF[0:6, :_E]
    w_t = EF[8:48, :_E]
    filt = FT[:, :_E].T

    edge_attri = jnp.einsum('re,we->erw', rr_t, w_t).reshape(_E, _NRBF, _NANG, _CH)
    rcv_p = ei_p[_E_PAD:]
    A0sc = _pass1(EF, rcv_p)
    A0 = (A0sc[:, :_N, :120].reshape(_NC, _N, _NRBF, _NANG, 2)
          .transpose(1, 2, 3, 0, 4).reshape(_N, _NRBF, _NANG, _CH))
    A = _radial_transform(A0, rt_w)
    B = _symmetrizer(A)
    mpn = 1.0 / _AVG ** 0.5

    memory = A * jnp.transpose(nm_w[_LOFA], (1, 0, 2))[None]
    Bs = jnp.einsum('rb,nrbc->nc', bchi_w, B,
                    precision=lax.Precision.HIGHEST)
    scal = Bs[snd]
    msg_bchi = scal[:, None, None, :] * edge_attri
    A_bchi = jnp.zeros((n_nodes, _NRBF, _NANG, _CH), positions.dtype).at[rcv].add(msg_bchi)
    A_bchi = _radial_transform(A_bchi, rt_w)
    msg_ar = A[snd] * filt[:, :, None, None]
    A_ar = jnp.zeros_like(A).at[rcv].add(msg_ar)
    A2 = (A_ar + A_bchi) * mpn + memory
    B2 = _symmetrizer(A2)
    node_feats = jnp.stack([B, B2], axis=-1)
    return node_feats


# speedup vs baseline: 14.7554x; 10.9229x over previous
import functools
from math import factorial

import jax
import jax.numpy as jnp
import numpy as np
from jax import lax
from jax.experimental import pallas as pl
from jax.experimental.pallas import tpu as pltpu
from jax.experimental.pallas import tpu_sc as plsc

_NZ = 4
_NAB = 2
_CH = _NAB * _NAB
_CUTOFF = 5.5
_NRBF = 6
_NRB = 8
_MAXL = 2
_AVG = 16.0
_LXLYLZ = [(0, 0, 0), (1, 0, 0), (0, 1, 0), (0, 0, 1), (2, 0, 0), (1, 1, 0),
           (1, 0, 1), (0, 2, 0), (0, 1, 1), (0, 0, 2)]
_NANG = len(_LXLYLZ)
_LOFA = np.array([lx + ly + lz for (lx, ly, lz) in _LXLYLZ])

_N = 10000
_E = 160000
_E_PAD = 163840
_NC = 2
_NS = 16
_NW = _NC * _NS
_EPW = _E_PAD // _NW

_mesh = plsc.VectorSubcoreMesh(core_axis_name="c", subcore_axis_name="s")


def _f16(v):
    return jnp.full((16,), v, jnp.int32)


def _p0_body(pos_hbm, emb_hbm, ei_hbm, ev_hbm, pos_v, emb_v, snd_v, rcv_v, out_v):
    cid = lax.axis_index("c")
    sid = lax.axis_index("s")
    w = sid * _NC + cid
    base = w * _EPW
    pltpu.sync_copy(pos_hbm, pos_v)
    pltpu.sync_copy(emb_hbm, emb_v)
    pltpu.sync_copy(ei_hbm.at[pl.ds(base, _EPW)], snd_v)
    pltpu.sync_copy(ei_hbm.at[pl.ds(_E_PAD + base, _EPW)], rcv_v)

    n1 = _f16(_N)
    n2 = _f16(2 * _N)

    def body(g, carry):
        j0 = g * 16
        snd16 = snd_v[pl.ds(j0, 16)]
        rcv16 = rcv_v[pl.ds(j0, 16)]
        pxs = plsc.load_gather(pos_v, [snd16])
        pys = plsc.load_gather(pos_v, [snd16 + n1])
        pzs = plsc.load_gather(pos_v, [snd16 + n2])
        pxr = plsc.load_gather(pos_v, [rcv16])
        pyr = plsc.load_gather(pos_v, [rcv16 + n1])
        pzr = plsc.load_gather(pos_v, [rcv16 + n2])
        es0 = plsc.load_gather(emb_v, [snd16])
        es1 = plsc.load_gather(emb_v, [snd16 + n1])
        er0 = plsc.load_gather(emb_v, [rcv16])
        er1 = plsc.load_gather(emb_v, [rcv16 + n1])
        out_v[pl.ds(0 * _EPW + j0, 16)] = pxr - pxs
        out_v[pl.ds(1 * _EPW + j0, 16)] = pyr - pys
        out_v[pl.ds(2 * _EPW + j0, 16)] = pzr - pzs
        out_v[pl.ds(3 * _EPW + j0, 16)] = es0 * er0
        out_v[pl.ds(4 * _EPW + j0, 16)] = es0 * er1
        out_v[pl.ds(5 * _EPW + j0, 16)] = es1 * er0
        out_v[pl.ds(6 * _EPW + j0, 16)] = es1 * er1
        return carry

    lax.fori_loop(0, _EPW // 16, body, 0)
    for comp in range(7):
        pltpu.sync_copy(out_v.at[pl.ds(comp * _EPW, _EPW)],
                        ev_hbm.at[pl.ds(comp * _E_PAD + base, _EPW)])


_sc_params = pltpu.CompilerParams(needs_layout_passes=False)

_p0 = functools.partial(
    pl.kernel,
    out_type=jax.ShapeDtypeStruct((7 * _E_PAD,), jnp.float32),
    mesh=_mesh,
    compiler_params=_sc_params,
    scratch_types=[
        pltpu.VMEM((3 * _N,), jnp.float32),
        pltpu.VMEM((2 * _N,), jnp.float32),
        pltpu.VMEM((_EPW,), jnp.int32),
        pltpu.VMEM((_EPW,), jnp.int32),
        pltpu.VMEM((7 * _EPW,), jnp.float32),
    ],
)(_p0_body)


_K = 256
_K2 = 128
_EPT = _E_PAD // _NS
_NCHUNK = _EPT // _K
_NCHUNK2 = _EPT // _K2
_N_PAD = 10240
_NPT = _N_PAD // _NS

_fl = np.arange(128)
_rpat_np = np.where(_fl < 120, _fl // 20, 6).astype(np.int32)
_wpat_np = np.where(_fl < 120, 8 + ((_fl % 20) // 2) * 4 + (_fl % 2), 6).astype(np.int32)
_wvalid_np = (_fl < 120).astype(np.int32)


def _pass1_body(ef_hbm, rcv_hbm, out_hbm, ef_v, rcv_v, rows_v, table):
    cid = lax.axis_index("c")
    tid = lax.axis_index("s")
    zf = jnp.zeros((16,), jnp.float32)

    def zrow(j, c):
        for i in range(8):
            rows_v[j, pl.ds(i * 16, 16)] = zf
        return c
    lax.fori_loop(0, 128, zrow, 0)
    for i in range(5):
        pltpu.sync_copy(rows_v.at[pl.ds(0, 128)],
                        table.at[pl.ds(tid * _NPT + i * 128, 128)])
    plsc.subcore_barrier()

    iota = lax.iota(jnp.int32, 16)
    rpats, wpats = [], []
    for i in range(8):
        fv = iota + 16 * i
        rpats.append(jnp.where(fv < 120, fv // 20, 6))
        wpats.append(jnp.where(fv < 120,
                               8 + ((fv % 20) // 2) * 4 + (fv % 2) + 2 * cid, 6))

    def chunk(ci, c):
        base = tid * _EPT + ci * _K
        pltpu.sync_copy(ef_hbm.at[:, pl.ds(base, _K)], ef_v)
        pltpu.sync_copy(rcv_hbm.at[pl.ds(base, _K)], rcv_v)

        def edge(j, c2):
            spl = jnp.full((16,), j, jnp.int32)
            for i in range(8):
                rrv = plsc.load_gather(ef_v, [rpats[i], spl])
                wv = plsc.load_gather(ef_v, [wpats[i], spl])
                rows_v[j, pl.ds(i * 16, 16)] = rrv * wv
            return c2
        lax.fori_loop(0, _K, edge, 0)
        pltpu.sync_copy(rows_v, table.at[rcv_v], add=True)
        return c
    lax.fori_loop(0, _NCHUNK, chunk, 0)
    plsc.subcore_barrier()
    pltpu.sync_copy(table.at[pl.ds(tid * _NPT, _NPT)],
                    out_hbm.at[cid, pl.ds(tid * _NPT, _NPT)])


_pass1 = functools.partial(
    pl.kernel,
    out_type=jax.ShapeDtypeStruct((_NC, _N_PAD, 128), jnp.float32),
    mesh=_mesh,
    compiler_params=_sc_params,
    scratch_types=[
        pltpu.VMEM((48, _K), jnp.float32),
        pltpu.VMEM((_K,), jnp.int32),
        pltpu.VMEM((_K, 128), jnp.float32),
        pltpu.VMEM_SHARED((_N_PAD, 128), jnp.float32),
    ],
)(_pass1_body)


def _pass2a_body(ef_hbm, rcv_hbm, snd_hbm, bs_hbm, out_hbm,
                 ef_v, rcv_v, snd_v, bs_v, scal_v, rows_v, table):
    cid = lax.axis_index("c")
    tid = lax.axis_index("s")
    zf = jnp.zeros((16,), jnp.float32)

    def zrow(j, c):
        for i in range(8):
            rows_v[j, pl.ds(i * 16, 16)] = zf
        return c
    lax.fori_loop(0, 128, zrow, 0)
    for i in range(5):
        pltpu.sync_copy(rows_v.at[pl.ds(0, 128)],
                        table.at[pl.ds(tid * _NPT + i * 128, 128)])
    pltpu.sync_copy(bs_hbm.at[pl.ds(cid * 2 * _N, 2 * _N)], bs_v)
    plsc.subcore_barrier()

    iota = lax.iota(jnp.int32, 16)
    rpats, wpats, ccpats = [], [], []
    for i in range(8):
        fv = iota + 16 * i
        rpats.append(jnp.where(fv < 120, fv // 20, 6))
        wpats.append(jnp.where(fv < 120,
                               8 + ((fv % 20) // 2) * 4 + (fv % 2) + 2 * cid, 6))
        ccpats.append(jnp.where(fv < 120, fv % 2, 0))

    def chunk(ci, c):
        base = tid * _EPT + ci * _K2
        pltpu.sync_copy(ef_hbm.at[:, pl.ds(base, _K2)], ef_v)
        pltpu.sync_copy(rcv_hbm.at[pl.ds(base, _K2)], rcv_v)
        pltpu.sync_copy(snd_hbm.at[pl.ds(base, _K2)], snd_v)

        def pregather(g, c2):
            j0 = g * 16
            i4 = snd_v[pl.ds(j0, 16)] * 2
            scal_v[0, pl.ds(j0, 16)] = plsc.load_gather(bs_v, [i4])
            scal_v[1, pl.ds(j0, 16)] = plsc.load_gather(bs_v, [i4 + 1])
            return c2
        lax.fori_loop(0, _K2 // 16, pregather, 0)

        def edge(j, c2):
            spl = jnp.full((16,), j, jnp.int32)
            for i in range(8):
                rrv = plsc.load_gather(ef_v, [rpats[i], spl])
                wv = plsc.load_gather(ef_v, [wpats[i], spl])
                sv = plsc.load_gather(scal_v, [ccpats[i], spl])
                rows_v[j, pl.ds(i * 16, 16)] = rrv * wv * sv
            return c2
        lax.fori_loop(0, _K2, edge, 0)
        pltpu.sync_copy(rows_v, table.at[rcv_v], add=True)
        return c
    lax.fori_loop(0, _NCHUNK2, chunk, 0)
    plsc.subcore_barrier()
    pltpu.sync_copy(table.at[pl.ds(tid * _NPT, _NPT)],
                    out_hbm.at[cid, pl.ds(tid * _NPT, _NPT)])


_pass2a = functools.partial(
    pl.kernel,
    out_type=jax.ShapeDtypeStruct((_NC, _N_PAD, 128), jnp.float32),
    mesh=_mesh,
    compiler_params=_sc_params,
    scratch_types=[
        pltpu.VMEM((48, _K2), jnp.float32),
        pltpu.VMEM((_K2,), jnp.int32),
        pltpu.VMEM((_K2,), jnp.int32),
        pltpu.VMEM((2 * _N,), jnp.float32),
        pltpu.VMEM((2, _K2), jnp.float32),
        pltpu.VMEM((_K2, 128), jnp.float32),
        pltpu.VMEM_SHARED((_N_PAD, 128), jnp.float32),
    ],
)(_pass2a_body)


def _make_pass2b(edge_split):
    nchunk = _NCHUNK2 // 2 if edge_split else _NCHUNK2

    def body(ft_hbm, rcv_hbm, snd_hbm, a_hbm, out_hbm,
             ft_v, rcv_v, snd_v, sadj_v, arow_v, rows_v, sem, table):
        cid = lax.axis_index("c")
        tid = lax.axis_index("s")
        zf = jnp.zeros((16,), jnp.float32)

        def zrow(j, c):
            for i in range(8):
                rows_v[j, pl.ds(i * 16, 16)] = zf
            return c
        lax.fori_loop(0, 128, zrow, 0)
        for i in range(5):
            pltpu.sync_copy(rows_v.at[pl.ds(0, 128)],
                            table.at[pl.ds(tid * _NPT + i * 128, 128)])
        plsc.subcore_barrier()

        iota = lax.iota(jnp.int32, 16)
        if edge_split:
            foff = 256
            coffn = jnp.full((16,), 0, jnp.int32)
        else:
            foff = cid * 128
            coffn = jnp.full((16,), cid * _N, jnp.int32)
        dpats = [jnp.minimum((foff + iota + 16 * i) // 40, 7) for i in range(8)]

        def chunk(ci, c):
            if edge_split:
                base = (cid * _NS + tid) * (_EPT // 2) + ci * _K2
            else:
                base = tid * _EPT + ci * _K2
            pltpu.sync_copy(ft_hbm.at[:, pl.ds(base, _K2)], ft_v)
            pltpu.sync_copy(rcv_hbm.at[pl.ds(base, _K2)], rcv_v)
            pltpu.sync_copy(snd_hbm.at[pl.ds(base, _K2)], snd_v)

            def adj(g, c2):
                j0 = g * 16
                sadj_v[pl.ds(j0, 16)] = snd_v[pl.ds(j0, 16)] + coffn
                return c2
            lax.fori_loop(0, _K2 // 16, adj, 0)
            pltpu.async_copy(a_hbm.at[sadj_v], arow_v, sem).wait()

            def edge(j, c2):
                spl = jnp.full((16,), j, jnp.int32)
                for i in range(8):
                    av = arow_v[j, pl.ds(i * 16, 16)]
                    fv = plsc.load_gather(ft_v, [dpats[i], spl])
                    rows_v[j, pl.ds(i * 16, 16)] = av * fv
                return c2
            lax.fori_loop(0, _K2, edge, 0)
            pltpu.sync_copy(rows_v, table.at[rcv_v], add=True)
            return c
        lax.fori_loop(0, nchunk, chunk, 0)
        plsc.subcore_barrier()
        pltpu.sync_copy(table.at[pl.ds(tid * _NPT, _NPT)],
                        out_hbm.at[cid, pl.ds(tid * _NPT, _NPT)])

    return functools.partial(
        pl.kernel,
        out_type=jax.ShapeDtypeStruct((_NC, _N_PAD, 128), jnp.float32),
        mesh=_mesh,
        compiler_params=_sc_params,
        scratch_types=[
            pltpu.VMEM((_NRB, _K2), jnp.float32),
            pltpu.VMEM((_K2,), jnp.int32),
            pltpu.VMEM((_K2,), jnp.int32),
            pltpu.VMEM((_K2,), jnp.int32),
            pltpu.VMEM((_K2, 128), jnp.float32),
            pltpu.VMEM((_K2, 128), jnp.float32),
            pltpu.SemaphoreType.DMA,
            pltpu.VMEM_SHARED((_N_PAD, 128), jnp.float32),
        ],
    )(body)


_pass2b_a = _make_pass2b(False)
_pass2b_b = _make_pass2b(True)


def _p1_body(ev_ref, arw_ref, ef_ref, ft_ref):
    ev = ev_ref[...]
    x, y, z = ev[0], ev[1], ev[2]
    enc = ev[3:7]
    l2 = x * x + y * y + z * z
    lengths = jnp.sqrt(l2)
    inv = 1.0 / (lengths + 1e-9)
    ux, uy, uz = x * inv, y * inv, z * inv

    r = lengths / _CUTOFF
    p = 6.0
    r6 = r ** 6
    env = (1.0 - 0.5 * (p + 1) * (p + 2) * r6 + p * (p + 2) * r6 * r
           - 0.5 * p * (p + 1) * r6 * r * r)
    rcut = env * (lengths < _CUTOFF).astype(jnp.float32)

    scale = jnp.sqrt(2.0 / _CUTOFF)
    bess = []
    for n in range(1, _NRB + 1):
        bess.append(scale * jnp.sin(n * jnp.pi * lengths / _CUTOFF) * inv)
    bess = jnp.stack(bess, axis=0)
    rr = bess[:_NRBF] * rcut[None, :]

    angs = []
    for (lx, ly, lz) in _LXLYLZ:
        v = jnp.ones_like(ux)
        for _ in range(lx):
            v = v * ux
        for _ in range(ly):
            v = v * uy
        for _ in range(lz):
            v = v * uz
        angs.append(v)
    ang = jnp.stack(angs, axis=0)
    wfe = (ang[:, None, :] * enc[None, :, :]).reshape(40, ang.shape[-1])

    ef_ref[...] = jnp.concatenate(
        [rr, jnp.zeros((2, rr.shape[-1]), jnp.float32), wfe], axis=0)

    arw = arw_ref[...]
    filt = lax.dot_general(arw, bess, (((0,), (0,)), ((), ())),
                           preferred_element_type=jnp.float32)
    ft_ref[...] = filt * rcut[None, :]


def _p1(ev, ar_w):
    BE = 512
    grid = (_E_PAD // BE,)
    return pl.pallas_call(
        _p1_body,
        grid=grid,
        in_specs=[
            pl.BlockSpec((7, BE), lambda i: (0, i)),
            pl.BlockSpec((_NRB, _NRB), lambda i: (0, 0)),
        ],
        out_specs=(
            pl.BlockSpec((48, BE), lambda i: (0, i)),
            pl.BlockSpec((_NRB, BE), lambda i: (0, i)),
        ),
        out_shape=(
            jax.ShapeDtypeStruct((48, _E_PAD), jnp.float32),
            jax.ShapeDtypeStruct((_NRB, _E_PAD), jnp.float32),
        ),
    )(ev, ar_w)


def _radial_transform(A, rt_w):
    return jnp.einsum('nrac,ard->ndac', A, rt_w[_LOFA])


def _symmetrizer(A):
    feats = [A[:, :, 0, :]]
    for l in range(_MAXL + 1):
        acc = jnp.zeros_like(A[:, :, 0, :])
        for a, (lx, ly, lz) in enumerate(_LXLYLZ):
            if lx + ly + lz == l:
                c = float(factorial(l) / (factorial(lx) * factorial(ly) * factorial(lz)))
                acc = acc + c * A[:, :, a, :] ** 2
        feats.append(acc)
    return jnp.stack(feats, axis=2)


def kernel(positions, atomic_numbers, edge_index, shifts, batch, cell,
           emb_w, rt_w, nm_w, ar_w, bchi_w):
    n_nodes = positions.shape[0]
    onehot = jax.nn.one_hot(atomic_numbers, _NZ, dtype=positions.dtype)
    node_emb = onehot @ emb_w

    ei_p = jnp.zeros((2, _E_PAD), jnp.int32).at[:, :_E].set(edge_index).reshape(-1)
    pos_t = positions.T.reshape(-1)
    emb_t = node_emb.T.reshape(-1)

    ev = _p0(pos_t, emb_t, ei_p).reshape(7, _E_PAD)
    EF, FT = _p1(ev, ar_w)

    snd_p = ei_p[:_E_PAD]
    rcv_p = ei_p[_E_PAD:]
    A0sc = _pass1(EF, rcv_p)
    A0 = (A0sc[:, :_N, :120].reshape(_NC, _N, _NRBF, _NANG, 2)
          .transpose(1, 2, 3, 0, 4).reshape(_N, _NRBF, _NANG, _CH))
    A = _radial_transform(A0, rt_w)
    B = _symmetrizer(A)
    mpn = 1.0 / _AVG ** 0.5

    memory = A * jnp.transpose(nm_w[_LOFA], (1, 0, 2))[None]
    Bs = jnp.einsum('rb,nrbc->nc', bchi_w, B,
                    precision=lax.Precision.HIGHEST)

    bs_sc = Bs.reshape(_N, 2, 2).transpose(1, 0, 2).reshape(-1)
    A0bsc = _pass2a(EF, rcv_p, snd_p, bs_sc)
    A_bchi = (A0bsc[:, :_N, :120].reshape(_NC, _N, _NRBF, _NANG, 2)
              .transpose(1, 2, 3, 0, 4).reshape(_N, _NRBF, _NANG, _CH))
    A_bchi = _radial_transform(A_bchi, rt_w)

    A_flat = A.reshape(_N, 320)
    dep_a = A0bsc[0, 0, 127] * 0.0
    A_u01 = (jnp.concatenate([A_flat[:, :128], A_flat[:, 128:256]], axis=0)
             + dep_a)
    AarA = _pass2b_a(FT, rcv_p, snd_p, A_u01)
    dep_b = AarA[0, 0, 127] * 0.0
    A_u2 = (jnp.zeros((_N, 128), jnp.float32).at[:, :64].set(A_flat[:, 256:])
            + dep_b)
    AarB = _pass2b_b(FT, rcv_p, snd_p, A_u2)
    A_ar = jnp.concatenate(
        [AarA[0, :_N], AarA[1, :_N],
         (AarB[0, :_N] + AarB[1, :_N])[:, :64]], axis=1).reshape(_N, _NRB, _NANG, _CH)

    A2 = (A_ar + A_bchi) * mpn + memory
    B2 = _symmetrizer(A2)
    node_feats = jnp.stack([B, B2], axis=-1)
    return node_feats
